# Initial kernel scaffold; baseline (speedup 1.0000x reference)
import functools

import jax
import jax.numpy as jnp
from jax import lax
from jax.experimental import pallas as pl
from jax.experimental.pallas import tpu as pltpu
from jax.experimental.pallas import tpu_sc as plsc

_NC = 2
_NS = 16
_NW = _NC * _NS
_L = 16


def _xwT_call(x, W, n, d_in, d_hid):
    bm = 2000

    def body(x_ref, w_ref, o_ref):
        xw = jnp.dot(x_ref[...], w_ref[...], preferred_element_type=jnp.float32)
        o_ref[...] = xw.T

    return pl.pallas_call(
        body,
        grid=(n // bm,),
        in_specs=[
            pl.BlockSpec((bm, d_in), lambda i: (i, 0)),
            pl.BlockSpec((d_in, d_hid), lambda i: (0, 0)),
        ],
        out_specs=pl.BlockSpec((d_hid, bm), lambda i: (0, i)),
        out_shape=jax.ShapeDtypeStruct((d_hid, n), jnp.float32),
    )(x, W)


def _sc_degree_call(dst, n, e):
    e_per_w = e // _NW
    mesh = plsc.VectorSubcoreMesh(
        core_axis_name="c", subcore_axis_name="s",
        num_cores=_NC, num_subcores=_NS)

    @functools.partial(
        pl.kernel,
        out_type=jax.ShapeDtypeStruct((_NW, n), jnp.float32),
        mesh=mesh,
        scratch_types=[
            pltpu.VMEM((e_per_w,), jnp.int32),
            pltpu.VMEM((n,), jnp.float32),
            pltpu.SemaphoreType.DMA,
        ],
    )
    def hist_kernel(dst_hbm, out_hbm, dst_v, hist_v, sem):
        wid = lax.axis_index("s") * _NC + lax.axis_index("c")
        base = wid * e_per_w
        cp = pltpu.async_copy(dst_hbm.at[pl.ds(base, e_per_w)], dst_v, sem)

        zeros = jnp.zeros((_L,), jnp.float32)

        @pl.loop(0, n, step=_L)
        def _(i):
            hist_v[pl.ds(i, _L)] = zeros

        cp.wait()
        ones = jnp.ones((_L,), jnp.float32)

        @pl.loop(0, e_per_w, step=_L)
        def _(i):
            d = dst_v[pl.ds(i, _L)]
            plsc.addupdate_scatter(hist_v, [d], ones)

        pltpu.sync_copy(hist_v, out_hbm.at[wid])

    return hist_kernel(dst)


def _dinv_y_call(deg_part, xwT, n, d_hid):
    def body(dp_ref, xwT_ref, y_ref, dinv_ref):
        deg = jnp.sum(dp_ref[...], axis=0) + 1.0
        dinv = lax.rsqrt(deg)
        dinv_ref[...] = dinv[None, :]
        y_ref[...] = xwT_ref[...] * dinv[None, :]

    bn = 2000
    return pl.pallas_call(
        body,
        grid=(n // bn,),
        in_specs=[
            pl.BlockSpec((_NW, bn), lambda i: (0, i)),
            pl.BlockSpec((d_hid, bn), lambda i: (0, i)),
        ],
        out_specs=[
            pl.BlockSpec((d_hid, bn), lambda i: (0, i)),
            pl.BlockSpec((1, bn), lambda i: (0, i)),
        ],
        out_shape=[
            jax.ShapeDtypeStruct((d_hid, n), jnp.float32),
            jax.ShapeDtypeStruct((1, n), jnp.float32),
        ],
    )(deg_part, xwT)


def _sc_msgpass_call(src, dst, y_flat, n, e, d_hid):
    e_per_w = e // _NW
    fn = d_hid * n
    mesh = plsc.VectorSubcoreMesh(
        core_axis_name="c", subcore_axis_name="s",
        num_cores=_NC, num_subcores=_NS)

    @functools.partial(
        pl.kernel,
        out_type=jax.ShapeDtypeStruct((_NW, fn), jnp.float32),
        mesh=mesh,
        scratch_types=[
            pltpu.VMEM((e_per_w,), jnp.int32),
            pltpu.VMEM((e_per_w,), jnp.int32),
            pltpu.VMEM((fn,), jnp.float32),
            pltpu.VMEM((fn,), jnp.float32),
            pltpu.SemaphoreType.DMA,
        ],
    )
    def msg_kernel(src_hbm, dst_hbm, y_hbm, out_hbm,
                   src_v, dst_v, y_v, agg_v, sem):
        wid = lax.axis_index("s") * _NC + lax.axis_index("c")
        base = wid * e_per_w
        cp1 = pltpu.async_copy(src_hbm.at[pl.ds(base, e_per_w)], src_v, sem)
        cp2 = pltpu.async_copy(dst_hbm.at[pl.ds(base, e_per_w)], dst_v, sem)
        cp3 = pltpu.async_copy(y_hbm, y_v, sem)

        zeros = jnp.zeros((_L,), jnp.float32)

        @pl.loop(0, fn, step=_L)
        def _(i):
            agg_v[pl.ds(i, _L)] = zeros

        cp1.wait()
        cp2.wait()
        cp3.wait()

        nvec = jnp.full((_L,), n, jnp.int32)

        @pl.loop(0, e_per_w, step=_L)
        def _(i):
            s0 = src_v[pl.ds(i, _L)]
            d0 = dst_v[pl.ds(i, _L)]
            s1 = s0 + nvec
            s2 = s1 + nvec
            d1 = d0 + nvec
            d2 = d1 + nvec
            v0 = plsc.load_gather(y_v, [s0])
            v1 = plsc.load_gather(y_v, [s1])
            v2 = plsc.load_gather(y_v, [s2])
            plsc.addupdate_scatter(agg_v, [d0], v0)
            plsc.addupdate_scatter(agg_v, [d1], v1)
            plsc.addupdate_scatter(agg_v, [d2], v2)

        pltpu.sync_copy(agg_v, out_hbm.at[wid])

    return msg_kernel(src, dst, y_flat)


def _final_call(agg_part3, y, dinv, b, W_lin, b_lin, n, d_hid, d_out):
    def body(ap_ref, y_ref, dinv_ref, b_ref, wl_ref, bl_ref, h_ref, z_ref):
        aggs = jnp.sum(ap_ref[...], axis=0)
        dinv = dinv_ref[...]
        aggc = (aggs + y_ref[...]) * dinv
        b_col = b_ref[...].reshape(d_hid, 1)
        h_cols = jnp.maximum(aggc + b_col, 0.0)
        z_cols = lax.dot_general(
            wl_ref[...], h_cols, (((0,), (0,)), ((), ())),
            preferred_element_type=jnp.float32)
        z_cols = z_cols + bl_ref[...].reshape(d_out, 1)
        h_ref[...] = h_cols.T
        z_ref[...] = z_cols.T

    bn = 2000
    return pl.pallas_call(
        body,
        grid=(n // bn,),
        in_specs=[
            pl.BlockSpec((_NW, d_hid, bn), lambda i: (0, 0, i)),
            pl.BlockSpec((d_hid, bn), lambda i: (0, i)),
            pl.BlockSpec((1, bn), lambda i: (0, i)),
            pl.BlockSpec((d_hid,), lambda i: (0,)),
            pl.BlockSpec((d_hid, d_out), lambda i: (0, 0)),
            pl.BlockSpec((d_out,), lambda i: (0,)),
        ],
        out_specs=[
            pl.BlockSpec((bn, d_hid), lambda i: (i, 0)),
            pl.BlockSpec((bn, d_out), lambda i: (i, 0)),
        ],
        out_shape=[
            jax.ShapeDtypeStruct((n, d_hid), jnp.float32),
            jax.ShapeDtypeStruct((n, d_out), jnp.float32),
        ],
    )(agg_part3, y, dinv, b, W_lin, b_lin)


def kernel(x, edges, W, b, W_lin, b_lin):
    n, d_in = x.shape
    d_hid = W.shape[1]
    d_out = W_lin.shape[1]
    e = edges.shape[1]
    assert e % (_NW * _L) == 0 and n % _L == 0

    src = edges[0].astype(jnp.int32)
    dst = edges[1].astype(jnp.int32)

    xwT = _xwT_call(x, W, n, d_in, d_hid)
    deg_part = _sc_degree_call(dst, n, e)
    y, dinv = _dinv_y_call(deg_part, xwT, n, d_hid)
    agg_part = _sc_msgpass_call(src, dst, y.reshape(-1), n, e, d_hid)
    agg_part3 = agg_part.reshape(_NW, d_hid, n)
    h, z = _final_call(agg_part3, y, dinv, b, W_lin, b_lin, n, d_hid, d_out)
    return (h, z)


# speedup vs baseline: 89.5917x; 89.5917x over previous
import dataclasses
import functools

import jax
import jax.numpy as jnp
from jax import lax
from jax.experimental import pallas as pl
from jax.experimental.pallas import tpu as pltpu
from jax.experimental.pallas import tpu_sc as plsc

_NC = 2
_NS = 16
_NW = _NC * _NS
_L = 16


def _sc_compiler_params():
    cp = pltpu.CompilerParams()
    if "needs_layout_passes" in pltpu.CompilerParams.__dataclass_fields__:
        cp = dataclasses.replace(cp, needs_layout_passes=False)
    return cp


def _xwT_call(x, W, n, d_in, d_hid):
    def body(x_ref, w_ref, o_ref):
        xw = jnp.dot(x_ref[...], w_ref[...], preferred_element_type=jnp.float32)
        o_ref[...] = xw.T

    return pl.pallas_call(
        body,
        out_shape=jax.ShapeDtypeStruct((d_hid, n), jnp.float32),
    )(x, W)


def _sc_degree_call(dst, n, e):
    e_per_w = e // _NW
    mesh = plsc.VectorSubcoreMesh(
        core_axis_name="c", subcore_axis_name="s",
        num_cores=_NC, num_subcores=_NS)

    @functools.partial(
        pl.kernel,
        out_type=jax.ShapeDtypeStruct((_NW, n), jnp.float32),
        mesh=mesh,
        scratch_types=[
            pltpu.VMEM((e_per_w,), jnp.int32),
            pltpu.VMEM((n,), jnp.float32),
            pltpu.SemaphoreType.DMA,
        ],
        compiler_params=_sc_compiler_params(),
    )
    def hist_kernel(dst_hbm, out_hbm, dst_v, hist_v, sem):
        wid = lax.axis_index("s") * _NC + lax.axis_index("c")
        base = wid * e_per_w
        cp = pltpu.async_copy(dst_hbm.at[pl.ds(base, e_per_w)], dst_v, sem)

        zeros = jnp.zeros((_L,), jnp.float32)

        @pl.loop(0, n, step=_L)
        def _(i):
            hist_v[pl.ds(i, _L)] = zeros

        cp.wait()
        ones = jnp.ones((_L,), jnp.float32)

        @pl.loop(0, e_per_w, step=_L)
        def _(i):
            d = dst_v[pl.ds(i, _L)]
            plsc.addupdate_scatter(hist_v, [d], ones)

        pltpu.sync_copy(hist_v, out_hbm.at[wid])

    return hist_kernel(dst)


def _dinv_y_call(deg_part, xwT, n, d_hid):
    def body(dp_ref, xwT_ref, y_ref, dinv_ref):
        deg = jnp.sum(dp_ref[...], axis=0) + 1.0
        dinv = lax.rsqrt(deg)
        dinv_ref[...] = dinv[None, :]
        y_ref[...] = xwT_ref[...] * dinv[None, :]

    return pl.pallas_call(
        body,
        out_shape=[
            jax.ShapeDtypeStruct((d_hid, n), jnp.float32),
            jax.ShapeDtypeStruct((1, n), jnp.float32),
        ],
    )(deg_part, xwT)


def _sc_msgpass_call(src, dst, y_flat, n, e, d_hid):
    e_per_w = e // _NW
    fn = d_hid * n
    mesh = plsc.VectorSubcoreMesh(
        core_axis_name="c", subcore_axis_name="s",
        num_cores=_NC, num_subcores=_NS)

    @functools.partial(
        pl.kernel,
        out_type=jax.ShapeDtypeStruct((_NW, fn), jnp.float32),
        mesh=mesh,
        scratch_types=[
            pltpu.VMEM((e_per_w,), jnp.int32),
            pltpu.VMEM((e_per_w,), jnp.int32),
            pltpu.VMEM((fn,), jnp.float32),
            pltpu.VMEM((fn,), jnp.float32),
            pltpu.SemaphoreType.DMA,
        ],
        compiler_params=_sc_compiler_params(),
    )
    def msg_kernel(src_hbm, dst_hbm, y_hbm, out_hbm,
                   src_v, dst_v, y_v, agg_v, sem):
        wid = lax.axis_index("s") * _NC + lax.axis_index("c")
        base = wid * e_per_w
        cp1 = pltpu.async_copy(src_hbm.at[pl.ds(base, e_per_w)], src_v, sem)
        cp2 = pltpu.async_copy(dst_hbm.at[pl.ds(base, e_per_w)], dst_v, sem)
        cp3 = pltpu.async_copy(y_hbm, y_v, sem)

        zeros = jnp.zeros((_L,), jnp.float32)

        @pl.loop(0, fn, step=_L)
        def _(i):
            agg_v[pl.ds(i, _L)] = zeros

        cp1.wait()
        cp2.wait()
        cp3.wait()

        nvec = jnp.full((_L,), n, jnp.int32)

        @pl.loop(0, e_per_w, step=_L)
        def _(i):
            s0 = src_v[pl.ds(i, _L)]
            d0 = dst_v[pl.ds(i, _L)]
            s1 = s0 + nvec
            s2 = s1 + nvec
            d1 = d0 + nvec
            d2 = d1 + nvec
            v0 = plsc.load_gather(y_v, [s0])
            v1 = plsc.load_gather(y_v, [s1])
            v2 = plsc.load_gather(y_v, [s2])
            plsc.addupdate_scatter(agg_v, [d0], v0)
            plsc.addupdate_scatter(agg_v, [d1], v1)
            plsc.addupdate_scatter(agg_v, [d2], v2)

        pltpu.sync_copy(agg_v, out_hbm.at[wid])

    return msg_kernel(src, dst, y_flat)


def _final_call(agg_part3, y, dinv, b, W_lin, b_lin, n, d_hid, d_out):
    def body(ap_ref, y_ref, dinv_ref, b_ref, wl_ref, bl_ref, h_ref, z_ref):
        aggs = jnp.sum(ap_ref[...], axis=0)
        dinv = dinv_ref[...]
        aggc = (aggs + y_ref[...]) * dinv
        b_col = b_ref[...].reshape(d_hid, 1)
        h_cols = jnp.maximum(aggc + b_col, 0.0)
        z_cols = lax.dot_general(
            wl_ref[...], h_cols, (((0,), (0,)), ((), ())),
            preferred_element_type=jnp.float32)
        z_cols = z_cols + bl_ref[...].reshape(d_out, 1)
        h_ref[...] = h_cols.T
        z_ref[...] = z_cols.T

    return pl.pallas_call(
        body,
        out_shape=[
            jax.ShapeDtypeStruct((n, d_hid), jnp.float32),
            jax.ShapeDtypeStruct((n, d_out), jnp.float32),
        ],
    )(agg_part3, y, dinv, b, W_lin, b_lin)


def kernel(x, edges, W, b, W_lin, b_lin):
    n, d_in = x.shape
    d_hid = W.shape[1]
    d_out = W_lin.shape[1]
    e = edges.shape[1]
    assert e % (_NW * _L) == 0 and n % _L == 0

    src = edges[0].astype(jnp.int32)
    dst = edges[1].astype(jnp.int32)

    xwT = _xwT_call(x, W, n, d_in, d_hid)
    deg_part = _sc_degree_call(dst, n, e)
    y, dinv = _dinv_y_call(deg_part, xwT, n, d_hid)
    agg_part = _sc_msgpass_call(src, dst, y.reshape(-1), n, e, d_hid)
    agg_part3 = agg_part.reshape(_NW, d_hid, n)
    h, z = _final_call(agg_part3, y, dinv, b, W_lin, b_lin, n, d_hid, d_out)
    return (h, z)


# speedup vs baseline: 94.3485x; 1.0531x over previous
import dataclasses
import functools

import jax
import jax.numpy as jnp
from jax import lax
from jax.experimental import pallas as pl
from jax.experimental.pallas import tpu as pltpu
from jax.experimental.pallas import tpu_sc as plsc

_NC = 2
_NS = 16
_NW = _NC * _NS
_L = 16


def _sc_compiler_params():
    cp = pltpu.CompilerParams()
    if "needs_layout_passes" in pltpu.CompilerParams.__dataclass_fields__:
        cp = dataclasses.replace(cp, needs_layout_passes=False)
    return cp


def _xwT_call(x, W, n, d_in, d_hid):
    def body(x_ref, w_ref, o_ref):
        xw = jnp.dot(x_ref[...], w_ref[...], preferred_element_type=jnp.float32)
        o_ref[...] = xw.T

    return pl.pallas_call(
        body,
        out_shape=jax.ShapeDtypeStruct((d_hid, n), jnp.float32),
    )(x, W)


def _sc_degree_call(dst, n, e):
    e_per_w = e // _NW
    mesh = plsc.VectorSubcoreMesh(
        core_axis_name="c", subcore_axis_name="s",
        num_cores=_NC, num_subcores=_NS)

    @functools.partial(
        pl.kernel,
        out_type=jax.ShapeDtypeStruct((_NW, n), jnp.float32),
        mesh=mesh,
        scratch_types=[
            pltpu.VMEM((e_per_w,), jnp.int32),
            pltpu.VMEM((n,), jnp.float32),
            pltpu.SemaphoreType.DMA,
        ],
        compiler_params=_sc_compiler_params(),
    )
    def hist_kernel(dst_hbm, out_hbm, dst_v, hist_v, sem):
        wid = lax.axis_index("s") * _NC + lax.axis_index("c")
        base = wid * e_per_w
        cp = pltpu.async_copy(dst_hbm.at[pl.ds(base, e_per_w)], dst_v, sem)

        zeros = jnp.zeros((_L,), jnp.float32)

        @pl.loop(0, n, step=_L, unroll=8)
        def _(i):
            hist_v[pl.ds(i, _L)] = zeros

        cp.wait()
        ones = jnp.ones((_L,), jnp.float32)

        @pl.loop(0, e_per_w, step=_L, unroll=8)
        def _(i):
            d = dst_v[pl.ds(i, _L)]
            plsc.addupdate_scatter(hist_v, [d], ones)

        pltpu.sync_copy(hist_v, out_hbm.at[wid])

    return hist_kernel(dst)


def _dinv_y_call(deg_part, xwT, n, d_hid):
    def body(dp_ref, xwT_ref, y_ref, dinv_ref):
        deg = jnp.sum(dp_ref[...], axis=0) + 1.0
        dinv = lax.rsqrt(deg)
        dinv_ref[...] = dinv[None, :]
        y_ref[...] = xwT_ref[...] * dinv[None, :]

    return pl.pallas_call(
        body,
        out_shape=[
            jax.ShapeDtypeStruct((d_hid, n), jnp.float32),
            jax.ShapeDtypeStruct((1, n), jnp.float32),
        ],
    )(deg_part, xwT)


def _sc_msgpass_call(src, dst, y_flat, n, e, d_hid):
    e_per_w = e // _NW
    fn = d_hid * n
    mesh = plsc.VectorSubcoreMesh(
        core_axis_name="c", subcore_axis_name="s",
        num_cores=_NC, num_subcores=_NS)

    @functools.partial(
        pl.kernel,
        out_type=jax.ShapeDtypeStruct((_NW, fn), jnp.float32),
        mesh=mesh,
        scratch_types=[
            pltpu.VMEM((e_per_w,), jnp.int32),
            pltpu.VMEM((e_per_w,), jnp.int32),
            pltpu.VMEM((fn,), jnp.float32),
            pltpu.VMEM((fn,), jnp.float32),
            pltpu.SemaphoreType.DMA,
        ],
        compiler_params=_sc_compiler_params(),
    )
    def msg_kernel(src_hbm, dst_hbm, y_hbm, out_hbm,
                   src_v, dst_v, y_v, agg_v, sem):
        wid = lax.axis_index("s") * _NC + lax.axis_index("c")
        base = wid * e_per_w
        cp1 = pltpu.async_copy(src_hbm.at[pl.ds(base, e_per_w)], src_v, sem)
        cp2 = pltpu.async_copy(dst_hbm.at[pl.ds(base, e_per_w)], dst_v, sem)
        cp3 = pltpu.async_copy(y_hbm, y_v, sem)

        zeros = jnp.zeros((_L,), jnp.float32)

        @pl.loop(0, fn, step=_L, unroll=8)
        def _(i):
            agg_v[pl.ds(i, _L)] = zeros

        cp1.wait()
        cp2.wait()
        cp3.wait()

        nvec = jnp.full((_L,), n, jnp.int32)

        @pl.loop(0, e_per_w, step=_L, unroll=4)
        def _(i):
            s0 = src_v[pl.ds(i, _L)]
            d0 = dst_v[pl.ds(i, _L)]
            s1 = s0 + nvec
            s2 = s1 + nvec
            d1 = d0 + nvec
            d2 = d1 + nvec
            v0 = plsc.load_gather(y_v, [s0])
            v1 = plsc.load_gather(y_v, [s1])
            v2 = plsc.load_gather(y_v, [s2])
            plsc.addupdate_scatter(agg_v, [d0], v0)
            plsc.addupdate_scatter(agg_v, [d1], v1)
            plsc.addupdate_scatter(agg_v, [d2], v2)

        pltpu.sync_copy(agg_v, out_hbm.at[wid])

    return msg_kernel(src, dst, y_flat)


def _final_call(agg_part3, y, dinv, b, W_lin, b_lin, n, d_hid, d_out):
    def body(ap_ref, y_ref, dinv_ref, b_ref, wl_ref, bl_ref, h_ref, z_ref):
        aggs = jnp.sum(ap_ref[...], axis=0)
        dinv = dinv_ref[...]
        aggc = (aggs + y_ref[...]) * dinv
        b_col = b_ref[...].reshape(d_hid, 1)
        h_cols = jnp.maximum(aggc + b_col, 0.0)
        z_cols = lax.dot_general(
            wl_ref[...], h_cols, (((0,), (0,)), ((), ())),
            preferred_element_type=jnp.float32)
        z_cols = z_cols + bl_ref[...].reshape(d_out, 1)
        h_ref[...] = h_cols.T
        z_ref[...] = z_cols.T

    return pl.pallas_call(
        body,
        out_shape=[
            jax.ShapeDtypeStruct((n, d_hid), jnp.float32),
            jax.ShapeDtypeStruct((n, d_out), jnp.float32),
        ],
    )(agg_part3, y, dinv, b, W_lin, b_lin)


def kernel(x, edges, W, b, W_lin, b_lin):
    n, d_in = x.shape
    d_hid = W.shape[1]
    d_out = W_lin.shape[1]
    e = edges.shape[1]
    assert e % (_NW * _L) == 0 and n % _L == 0

    src = edges[0].astype(jnp.int32)
    dst = edges[1].astype(jnp.int32)

    xwT = _xwT_call(x, W, n, d_in, d_hid)
    deg_part = _sc_degree_call(dst, n, e)
    y, dinv = _dinv_y_call(deg_part, xwT, n, d_hid)
    agg_part = _sc_msgpass_call(src, dst, y.reshape(-1), n, e, d_hid)
    agg_part3 = agg_part.reshape(_NW, d_hid, n)
    h, z = _final_call(agg_part3, y, dinv, b, W_lin, b_lin, n, d_hid, d_out)
    return (h, z)


# speedup vs baseline: 127.8794x; 1.3554x over previous
import dataclasses
import functools

import jax
import jax.numpy as jnp
from jax import lax
from jax.experimental import pallas as pl
from jax.experimental.pallas import tpu as pltpu
from jax.experimental.pallas import tpu_sc as plsc

_NC = 2
_NS = 16
_NW = _NC * _NS
_L = 16


def _sc_compiler_params():
    cp = pltpu.CompilerParams()
    if "needs_layout_passes" in pltpu.CompilerParams.__dataclass_fields__:
        cp = dataclasses.replace(cp, needs_layout_passes=False)
    return cp


def _xwT_call(x, W, n, d_in, d_hid):
    def body(x_ref, w_ref, o_ref):
        xw = jnp.dot(x_ref[...], w_ref[...], preferred_element_type=jnp.float32)
        o_ref[...] = xw.T

    return pl.pallas_call(
        body,
        out_shape=jax.ShapeDtypeStruct((d_hid, n), jnp.float32),
    )(x, W)


def _sc_degree_call(edges, n, e):
    e_per_w = e // _NW
    mesh = plsc.VectorSubcoreMesh(
        core_axis_name="c", subcore_axis_name="s",
        num_cores=_NC, num_subcores=_NS)

    @functools.partial(
        pl.kernel,
        out_type=jax.ShapeDtypeStruct((_NW, n), jnp.float32),
        mesh=mesh,
        scratch_types=[
            pltpu.VMEM((e_per_w,), jnp.int32),
            pltpu.VMEM((n,), jnp.float32),
            pltpu.SemaphoreType.DMA,
        ],
        compiler_params=_sc_compiler_params(),
    )
    def hist_kernel(edges_hbm, out_hbm, dst_v, hist_v, sem):
        wid = lax.axis_index("s") * _NC + lax.axis_index("c")
        base = e + wid * e_per_w
        cp = pltpu.async_copy(edges_hbm.at[pl.ds(base, e_per_w)], dst_v, sem)

        zeros = jnp.zeros((_L,), jnp.float32)

        @pl.loop(0, n, step=_L, unroll=8)
        def _(i):
            hist_v[pl.ds(i, _L)] = zeros

        cp.wait()
        ones = jnp.ones((_L,), jnp.float32)

        @pl.loop(0, e_per_w, step=_L, unroll=8)
        def _(i):
            d = dst_v[pl.ds(i, _L)]
            plsc.addupdate_scatter(hist_v, [d], ones)

        pltpu.sync_copy(hist_v, out_hbm.at[wid])

    return hist_kernel(edges)


def _dinv_y_call(deg_part, xwT, n, d_hid):
    def body(dp_ref, xwT_ref, y_ref, dinv_ref):
        deg = jnp.sum(dp_ref[...], axis=0) + 1.0
        dinv = lax.rsqrt(deg)
        dinv_ref[...] = dinv[None, :]
        y_ref[...] = xwT_ref[...] * dinv[None, :]

    return pl.pallas_call(
        body,
        out_shape=[
            jax.ShapeDtypeStruct((d_hid, n), jnp.float32),
            jax.ShapeDtypeStruct((1, n), jnp.float32),
        ],
    )(deg_part, xwT)


def _sc_msgpass_call(edges, y_flat, n, e, d_hid):
    e_per_w = e // _NW
    fn = d_hid * n
    mesh = plsc.VectorSubcoreMesh(
        core_axis_name="c", subcore_axis_name="s",
        num_cores=_NC, num_subcores=_NS)

    @functools.partial(
        pl.kernel,
        out_type=jax.ShapeDtypeStruct((_NW, fn), jnp.float32),
        mesh=mesh,
        scratch_types=[
            pltpu.VMEM((e_per_w,), jnp.int32),
            pltpu.VMEM((e_per_w,), jnp.int32),
            pltpu.VMEM((fn,), jnp.float32),
            pltpu.VMEM((fn,), jnp.float32),
            pltpu.SemaphoreType.DMA,
        ],
        compiler_params=_sc_compiler_params(),
    )
    def msg_kernel(edges_hbm, y_hbm, out_hbm,
                   src_v, dst_v, y_v, agg_v, sem):
        wid = lax.axis_index("s") * _NC + lax.axis_index("c")
        base = wid * e_per_w
        cp1 = pltpu.async_copy(edges_hbm.at[pl.ds(base, e_per_w)], src_v, sem)
        cp2 = pltpu.async_copy(edges_hbm.at[pl.ds(e + base, e_per_w)], dst_v, sem)
        cp3 = pltpu.async_copy(y_hbm, y_v, sem)

        zeros = jnp.zeros((_L,), jnp.float32)

        @pl.loop(0, fn, step=_L, unroll=8)
        def _(i):
            agg_v[pl.ds(i, _L)] = zeros

        cp1.wait()
        cp2.wait()
        cp3.wait()

        nvec = jnp.full((_L,), n, jnp.int32)

        @pl.loop(0, e_per_w, step=_L, unroll=4)
        def _(i):
            s0 = src_v[pl.ds(i, _L)]
            d0 = dst_v[pl.ds(i, _L)]
            s1 = s0 + nvec
            s2 = s1 + nvec
            d1 = d0 + nvec
            d2 = d1 + nvec
            v0 = plsc.load_gather(y_v, [s0])
            v1 = plsc.load_gather(y_v, [s1])
            v2 = plsc.load_gather(y_v, [s2])
            plsc.addupdate_scatter(agg_v, [d0], v0)
            plsc.addupdate_scatter(agg_v, [d1], v1)
            plsc.addupdate_scatter(agg_v, [d2], v2)

        pltpu.sync_copy(agg_v, out_hbm.at[wid])

    return msg_kernel(edges, y_flat)


def _final_call(agg_part3, y, dinv, b, W_lin, b_lin, n, d_hid, d_out):
    def body(ap_ref, y_ref, dinv_ref, b_ref, wl_ref, bl_ref, h_ref, z_ref):
        aggs = jnp.sum(ap_ref[...], axis=0)
        dinv = dinv_ref[...]
        aggc = (aggs + y_ref[...]) * dinv
        b_col = b_ref[...].reshape(d_hid, 1)
        h_cols = jnp.maximum(aggc + b_col, 0.0)
        z_cols = lax.dot_general(
            wl_ref[...], h_cols, (((0,), (0,)), ((), ())),
            preferred_element_type=jnp.float32)
        z_cols = z_cols + bl_ref[...].reshape(d_out, 1)
        h_ref[...] = h_cols
        z_ref[...] = z_cols

    return pl.pallas_call(
        body,
        out_shape=[
            jax.ShapeDtypeStruct((d_hid, n), jnp.float32),
            jax.ShapeDtypeStruct((d_out, n), jnp.float32),
        ],
    )(agg_part3, y, dinv, b, W_lin, b_lin)


def kernel(x, edges, W, b, W_lin, b_lin):
    n, d_in = x.shape
    d_hid = W.shape[1]
    d_out = W_lin.shape[1]
    e = edges.shape[1]
    assert e % (_NW * _L) == 0 and n % _L == 0

    edges = edges.astype(jnp.int32).reshape(-1)

    xwT = _xwT_call(x, W, n, d_in, d_hid)
    deg_part = _sc_degree_call(edges, n, e)
    y, dinv = _dinv_y_call(deg_part, xwT, n, d_hid)
    agg_part = _sc_msgpass_call(edges, y.reshape(-1), n, e, d_hid)
    agg_part3 = agg_part.reshape(_NW, d_hid, n)
    h_cols, z_cols = _final_call(agg_part3, y, dinv, b, W_lin, b_lin,
                                 n, d_hid, d_out)
    return (h_cols.T, z_cols.T)


# speedup vs baseline: 135.6831x; 1.0610x over previous
import dataclasses
import functools

import jax
import jax.numpy as jnp
from jax import lax
from jax.experimental import pallas as pl
from jax.experimental.pallas import tpu as pltpu
from jax.experimental.pallas import tpu_sc as plsc

_NC = 2
_NS = 16
_NW = _NC * _NS
_L = 16


def _sc_compiler_params():
    cp = pltpu.CompilerParams()
    if "needs_layout_passes" in pltpu.CompilerParams.__dataclass_fields__:
        cp = dataclasses.replace(cp, needs_layout_passes=False)
    return cp


def _xwT_call(x, W, n, d_in, d_hid):
    def body(x_ref, w_ref, o_ref):
        xw = jnp.dot(x_ref[...], w_ref[...], preferred_element_type=jnp.float32)
        o_ref[...] = xw.T

    return pl.pallas_call(
        body,
        out_shape=jax.ShapeDtypeStruct((d_hid, n), jnp.float32),
    )(x, W)


def _edge_chunk(e):
    tiles128 = e // 128
    base_t = tiles128 // _NW
    rem = tiles128 - base_t * _NW
    return base_t * 128, rem


def _sc_degree_call(edges2, n, e):
    base_len, rem = _edge_chunk(e)
    big_len = base_len + 128
    mesh = plsc.VectorSubcoreMesh(
        core_axis_name="c", subcore_axis_name="s",
        num_cores=_NC, num_subcores=_NS)

    @functools.partial(
        pl.kernel,
        out_type=jax.ShapeDtypeStruct((_NW, n), jnp.float32),
        mesh=mesh,
        scratch_types=[
            pltpu.VMEM((2, big_len), jnp.int32),
            pltpu.VMEM((n,), jnp.float32),
            pltpu.SemaphoreType.DMA,
        ],
        compiler_params=_sc_compiler_params(),
    )
    def hist_kernel(edges_hbm, out_hbm, slab_v, hist_v, sem):
        wid = lax.axis_index("s") * _NC + lax.axis_index("c")
        base = wid * base_len + jnp.minimum(wid, rem) * 128
        zeros = jnp.zeros((_L,), jnp.float32)
        ones = jnp.ones((_L,), jnp.float32)

        def work(clen):
            cp = pltpu.async_copy(
                edges_hbm.at[:, pl.ds(base, clen)],
                slab_v.at[:, pl.ds(0, clen)], sem)

            @pl.loop(0, n, step=_L, unroll=8)
            def _(i):
                hist_v[pl.ds(i, _L)] = zeros

            cp.wait()

            @pl.loop(0, clen, step=_L, unroll=8)
            def _(i):
                d = slab_v[1, pl.ds(i, _L)]
                plsc.addupdate_scatter(hist_v, [d], ones)

        @pl.when(wid < rem)
        def _():
            work(big_len)

        @pl.when(wid >= rem)
        def _():
            work(base_len)

        pltpu.sync_copy(hist_v, out_hbm.at[wid])

    return hist_kernel(edges2)


def _dinv_y_call(deg_part, xwT, n, d_hid):
    def body(dp_ref, xwT_ref, y_ref, yflat_ref, dinv_ref):
        deg = jnp.sum(dp_ref[...], axis=0) + 1.0
        dinv = lax.rsqrt(deg)
        dinv_ref[...] = dinv[None, :]
        y = xwT_ref[...] * dinv[None, :]
        y_ref[...] = y
        for c in range(d_hid):
            yflat_ref[pl.ds(c * n, n)] = y[c]

    return pl.pallas_call(
        body,
        out_shape=[
            jax.ShapeDtypeStruct((d_hid, n), jnp.float32),
            jax.ShapeDtypeStruct((d_hid * n,), jnp.float32),
            jax.ShapeDtypeStruct((1, n), jnp.float32),
        ],
    )(deg_part, xwT)


def _sc_msgpass_call(edges2, y_flat, n, e, d_hid):
    base_len, rem = _edge_chunk(e)
    big_len = base_len + 128
    fn = d_hid * n
    mesh = plsc.VectorSubcoreMesh(
        core_axis_name="c", subcore_axis_name="s",
        num_cores=_NC, num_subcores=_NS)

    @functools.partial(
        pl.kernel,
        out_type=jax.ShapeDtypeStruct((_NW, fn), jnp.float32),
        mesh=mesh,
        scratch_types=[
            pltpu.VMEM((2, big_len), jnp.int32),
            pltpu.VMEM((fn,), jnp.float32),
            pltpu.VMEM((fn,), jnp.float32),
            pltpu.SemaphoreType.DMA,
        ],
        compiler_params=_sc_compiler_params(),
    )
    def msg_kernel(edges_hbm, y_hbm, out_hbm,
                   slab_v, y_v, agg_v, sem):
        wid = lax.axis_index("s") * _NC + lax.axis_index("c")
        base = wid * base_len + jnp.minimum(wid, rem) * 128
        zeros = jnp.zeros((_L,), jnp.float32)
        nvec = jnp.full((_L,), n, jnp.int32)

        def work(clen):
            cp1 = pltpu.async_copy(
                edges_hbm.at[:, pl.ds(base, clen)],
                slab_v.at[:, pl.ds(0, clen)], sem)
            cp3 = pltpu.async_copy(y_hbm, y_v, sem)

            @pl.loop(0, fn, step=_L, unroll=8)
            def _(i):
                agg_v[pl.ds(i, _L)] = zeros

            cp1.wait()
            cp3.wait()

            @pl.loop(0, clen, step=_L, unroll=4)
            def _(i):
                s0 = slab_v[0, pl.ds(i, _L)]
                d0 = slab_v[1, pl.ds(i, _L)]
                s1 = s0 + nvec
                s2 = s1 + nvec
                d1 = d0 + nvec
                d2 = d1 + nvec
                v0 = plsc.load_gather(y_v, [s0])
                v1 = plsc.load_gather(y_v, [s1])
                v2 = plsc.load_gather(y_v, [s2])
                plsc.addupdate_scatter(agg_v, [d0], v0)
                plsc.addupdate_scatter(agg_v, [d1], v1)
                plsc.addupdate_scatter(agg_v, [d2], v2)

        @pl.when(wid < rem)
        def _():
            work(big_len)

        @pl.when(wid >= rem)
        def _():
            work(base_len)

        pltpu.sync_copy(agg_v, out_hbm.at[wid])

    return msg_kernel(edges2, y_flat)


def _final_call(agg_part3, y, dinv, b, W_lin, b_lin, n, d_hid, d_out):
    def body(ap_ref, y_ref, dinv_ref, b_ref, wl_ref, bl_ref, h_ref, z_ref):
        aggs = jnp.sum(ap_ref[...], axis=0)
        dinv = dinv_ref[...]
        aggc = (aggs + y_ref[...]) * dinv
        b_col = b_ref[...].reshape(d_hid, 1)
        h_cols = jnp.maximum(aggc + b_col, 0.0)
        z_cols = lax.dot_general(
            wl_ref[...], h_cols, (((0,), (0,)), ((), ())),
            preferred_element_type=jnp.float32)
        z_cols = z_cols + bl_ref[...].reshape(d_out, 1)
        h_ref[...] = h_cols
        z_ref[...] = z_cols

    return pl.pallas_call(
        body,
        out_shape=[
            jax.ShapeDtypeStruct((d_hid, n), jnp.float32),
            jax.ShapeDtypeStruct((d_out, n), jnp.float32),
        ],
    )(agg_part3, y, dinv, b, W_lin, b_lin)


def kernel(x, edges, W, b, W_lin, b_lin):
    n, d_in = x.shape
    d_hid = W.shape[1]
    d_out = W_lin.shape[1]
    e = edges.shape[1]
    assert e % (_NW * _L) == 0 and n % _L == 0

    edges = edges.astype(jnp.int32)

    xwT = _xwT_call(x, W, n, d_in, d_hid)
    deg_part = _sc_degree_call(edges, n, e)
    y, y_flat, dinv = _dinv_y_call(deg_part, xwT, n, d_hid)
    agg_part = _sc_msgpass_call(edges, y_flat, n, e, d_hid)
    agg_part3 = agg_part.reshape(_NW, d_hid, n)
    h_cols, z_cols = _final_call(agg_part3, y, dinv, b, W_lin, b_lin,
                                 n, d_hid, d_out)
    return (h_cols.T, z_cols.T)
